# baseline (speedup 1.0000x reference)
import functools

import jax
import jax.numpy as jnp
from jax import lax
from jax.experimental import pallas as pl
from jax.experimental.pallas import tpu as pltpu
from jax.experimental.pallas import tpu_sc as plsc

N_NODES = 10000
N_EDGES = 320000
D = 128

NC = 2
NS = 16
L = 16

CHUNK = 128
NCH = 80
N_CHUNKS = NC * NS * NCH
E_PAD = N_CHUNKS * CHUNK

NSTG = 40

ROWS_PER_TILE = 624
REM_BASE = ROWS_PER_TILE * NS
REM_ROWS = N_NODES - REM_BASE

_mesh = plsc.VectorSubcoreMesh(core_axis_name="c", subcore_axis_name="s")


@functools.partial(
    pl.kernel,
    out_type=jax.ShapeDtypeStruct((NC, N_NODES, D), jnp.float32),
    mesh=_mesh,
    compiler_params=pltpu.CompilerParams(needs_layout_passes=False),
    scratch_types=[
        pltpu.VMEM_SHARED((N_NODES, D), jnp.float32),
        pltpu.VMEM((NSTG, CHUNK), jnp.int32),
        pltpu.VMEM((NSTG, CHUNK), jnp.int32),
        pltpu.VMEM((NSTG, CHUNK), jnp.float32),
        pltpu.VMEM((CHUNK, D), jnp.float32),
        pltpu.VMEM((CHUNK, D), jnp.float32),
        pltpu.SemaphoreType.DMA,
        pltpu.SemaphoreType.DMA,
    ],
)
def _sc_agg(h_hbm, src_hbm, dst_hbm, w_hbm, zeros_hbm, out_hbm,
            acc, srcs, dsts, ws, rb0, rb1, sg0, sg1):
    cid = lax.axis_index("c")
    sid = lax.axis_index("s")
    wid = sid * NC + cid
    first = wid * NCH

    pltpu.sync_copy(zeros_hbm.at[pl.ds(sid * ROWS_PER_TILE, ROWS_PER_TILE)],
                    acc.at[pl.ds(sid * ROWS_PER_TILE, ROWS_PER_TILE)])

    @pl.when(sid == NS - 1)
    def _zero_rem():
        pltpu.sync_copy(zeros_hbm.at[pl.ds(REM_BASE, REM_ROWS)],
                        acc.at[pl.ds(REM_BASE, REM_ROWS)])

    plsc.subcore_barrier()

    def gather_start(g, rb, sem):
        pltpu.async_copy(h_hbm.at[srcs.at[g]], rb, sem)

    def gather_wait(g, rb, sem):
        pltpu.make_async_copy(h_hbm.at[srcs.at[g]], rb, sem).wait()

    def scale(g, rb):
        def body(e, c2):
            w16 = plsc.load_gather(
                ws, [jnp.broadcast_to(g, (L,)), jnp.broadcast_to(e, (L,))])
            for j in range(D // L):
                sl = pl.ds(j * L, L)
                rb[e, sl] = rb[e, sl] * w16
            return c2

        lax.fori_loop(0, CHUNK, body, 0, unroll=2)

    def scatter(g, rb):
        pltpu.sync_copy(rb, acc.at[dsts.at[g]], add=True)

    for s in range(NCH // NSTG):
        pltpu.sync_copy(src_hbm.at[pl.ds(first + s * NSTG, NSTG)], srcs)
        pltpu.sync_copy(dst_hbm.at[pl.ds(first + s * NSTG, NSTG)], dsts)
        pltpu.sync_copy(w_hbm.at[pl.ds(first + s * NSTG, NSTG)], ws)

        gather_start(0, rb0, sg0)

        def block(j, carry):
            c0 = 2 * j
            gather_start(c0 + 1, rb1, sg1)
            gather_wait(c0, rb0, sg0)
            scale(c0, rb0)
            scatter(c0, rb0)
            gather_start(c0 + 2, rb0, sg0)
            gather_wait(c0 + 1, rb1, sg1)
            scale(c0 + 1, rb1)
            scatter(c0 + 1, rb1)
            return carry

        lax.fori_loop(0, NSTG // 2 - 1, block, 0)

        gather_start(NSTG - 1, rb1, sg1)
        gather_wait(NSTG - 2, rb0, sg0)
        scale(NSTG - 2, rb0)
        scatter(NSTG - 2, rb0)
        gather_wait(NSTG - 1, rb1, sg1)
        scale(NSTG - 1, rb1)
        scatter(NSTG - 1, rb1)

    plsc.subcore_barrier()

    pltpu.sync_copy(acc.at[pl.ds(sid * ROWS_PER_TILE, ROWS_PER_TILE)],
                    out_hbm.at[cid, pl.ds(sid * ROWS_PER_TILE, ROWS_PER_TILE)])

    @pl.when(sid == NS - 1)
    def _write_rem():
        pltpu.sync_copy(acc.at[pl.ds(REM_BASE, REM_ROWS)],
                        out_hbm.at[cid, pl.ds(REM_BASE, REM_ROWS)])


_BLK = 1000


def _tc_body(p_ref, h_ref, wrel_ref, wroot_ref, b_ref, o_ref):
    agg = p_ref[0] + p_ref[1]
    o_ref[...] = (
        jnp.dot(agg, wrel_ref[...], preferred_element_type=jnp.float32)
        + jnp.dot(h_ref[...], wroot_ref[...], preferred_element_type=jnp.float32)
        + b_ref[...]
    )


_tc_combine = pl.pallas_call(
    _tc_body,
    grid=(N_NODES // _BLK,),
    in_specs=[
        pl.BlockSpec((NC, _BLK, D), lambda i: (0, i, 0)),
        pl.BlockSpec((_BLK, D), lambda i: (i, 0)),
        pl.BlockSpec((D, D), lambda i: (0, 0)),
        pl.BlockSpec((D, D), lambda i: (0, 0)),
        pl.BlockSpec((1, D), lambda i: (0, 0)),
    ],
    out_specs=pl.BlockSpec((_BLK, D), lambda i: (i, 0)),
    out_shape=jax.ShapeDtypeStruct((N_NODES, D), jnp.float32),
)


def kernel(x, edge_index, edge_attr, W_rel1, b_rel1, W_root1,
           W_rel2, b_rel2, W_root2):
    pad = E_PAD - N_EDGES
    src = jnp.concatenate([edge_index[0], jnp.zeros((pad,), jnp.int32)])
    dst = jnp.concatenate([edge_index[1], jnp.zeros((pad,), jnp.int32)])
    w = jnp.concatenate([edge_attr, jnp.zeros((pad,), jnp.float32)])
    src2 = src.reshape(N_CHUNKS, CHUNK)
    dst2 = dst.reshape(N_CHUNKS, CHUNK)
    w2 = w.reshape(N_CHUNKS, CHUNK)
    zeros = jnp.zeros((N_NODES, D), jnp.float32)

    p1 = _sc_agg(x, src2, dst2, w2, zeros)
    h1 = _tc_combine(p1, x, W_rel1, W_root1, b_rel1.reshape(1, D))
    p2 = _sc_agg(h1, src2, dst2, w2, zeros)
    h2 = _tc_combine(p2, h1, W_rel2, W_root2, b_rel2.reshape(1, D))
    return h2


# speedup vs baseline: 1.5062x; 1.5062x over previous
import functools

import jax
import jax.numpy as jnp
from jax import lax
from jax.experimental import pallas as pl
from jax.experimental.pallas import tpu as pltpu
from jax.experimental.pallas import tpu_sc as plsc

N_NODES = 10000
N_EDGES = 320000
D = 128

NC = 2
NS = 16
L = 16

CHUNK = 128
N_CHUNKS = N_EDGES // CHUNK
ROWS_PER_TILE = 624
REM_BASE = ROWS_PER_TILE * NS
REM_ROWS = N_NODES - REM_BASE

_mesh = plsc.VectorSubcoreMesh(core_axis_name="c", subcore_axis_name="s")


@functools.partial(
    pl.kernel,
    out_type=jax.ShapeDtypeStruct((NC, N_NODES, D), jnp.float32),
    mesh=_mesh,
    compiler_params=pltpu.CompilerParams(needs_layout_passes=False),
    scratch_types=[
        pltpu.VMEM_SHARED((N_NODES, D), jnp.float32),
        pltpu.VMEM((CHUNK,), jnp.int32),
        pltpu.VMEM((CHUNK,), jnp.int32),
        pltpu.VMEM((CHUNK,), jnp.float32),
        pltpu.VMEM((CHUNK, D), jnp.float32),
        pltpu.SemaphoreType.DMA,
        pltpu.SemaphoreType.DMA,
        pltpu.SemaphoreType.DMA,
        pltpu.SemaphoreType.DMA,
    ],
)
def _sc_agg(h_hbm, src_hbm, dst_hbm, w_hbm, zeros_hbm, out_hbm,
            acc, src_v, dst_v, w_v, rows_v, sem, s_src, s_dst, s_w):
    cid = lax.axis_index("c")
    sid = lax.axis_index("s")
    wid = sid * NC + cid

    pltpu.sync_copy(zeros_hbm.at[pl.ds(sid * ROWS_PER_TILE, ROWS_PER_TILE)],
                    acc.at[pl.ds(sid * ROWS_PER_TILE, ROWS_PER_TILE)])

    @pl.when(sid == NS - 1)
    def _zero_rem():
        pltpu.sync_copy(zeros_hbm.at[pl.ds(REM_BASE, REM_ROWS)],
                        acc.at[pl.ds(REM_BASE, REM_ROWS)])

    plsc.subcore_barrier()

    n_base = N_CHUNKS // (NC * NS)
    n_rem = N_CHUNKS % (NC * NS)
    my_n = jnp.where(wid < n_rem, n_base + 1, n_base)
    my_start = wid * n_base + jnp.minimum(wid, n_rem)

    def chunk_body(g, carry):
        base = (my_start + g) * CHUNK
        d_src = pltpu.async_copy(src_hbm.at[pl.ds(base, CHUNK)], src_v, s_src)
        d_dst = pltpu.async_copy(dst_hbm.at[pl.ds(base, CHUNK)], dst_v, s_dst)
        d_w = pltpu.async_copy(w_hbm.at[pl.ds(base, CHUNK)], w_v, s_w)
        d_src.wait()
        d_rows = pltpu.async_copy(h_hbm.at[src_v], rows_v, sem)
        d_dst.wait()
        d_w.wait()
        d_rows.wait()

        def scale_body(e, c2):
            w16 = plsc.load_gather(w_v, [jnp.broadcast_to(e, (L,))])
            for j in range(D // L):
                sl = pl.ds(j * L, L)
                rows_v[e, sl] = rows_v[e, sl] * w16
            return c2

        lax.fori_loop(0, CHUNK, scale_body, 0, unroll=2)
        pltpu.sync_copy(rows_v, acc.at[dst_v], add=True)
        return carry

    lax.fori_loop(0, my_n, chunk_body, 0)
    plsc.subcore_barrier()

    pltpu.sync_copy(acc.at[pl.ds(sid * ROWS_PER_TILE, ROWS_PER_TILE)],
                    out_hbm.at[cid, pl.ds(sid * ROWS_PER_TILE, ROWS_PER_TILE)])

    @pl.when(sid == NS - 1)
    def _write_rem():
        pltpu.sync_copy(acc.at[pl.ds(REM_BASE, REM_ROWS)],
                        out_hbm.at[cid, pl.ds(REM_BASE, REM_ROWS)])


_BLK = 1000


def _tc_body(p_ref, h_ref, wrel_ref, wroot_ref, b_ref, o_ref):
    agg = p_ref[0] + p_ref[1]
    o_ref[...] = (
        jnp.dot(agg, wrel_ref[...], preferred_element_type=jnp.float32)
        + jnp.dot(h_ref[...], wroot_ref[...], preferred_element_type=jnp.float32)
        + b_ref[...]
    )


_tc_combine = pl.pallas_call(
    _tc_body,
    grid=(N_NODES // _BLK,),
    in_specs=[
        pl.BlockSpec((NC, _BLK, D), lambda i: (0, i, 0)),
        pl.BlockSpec((_BLK, D), lambda i: (i, 0)),
        pl.BlockSpec((D, D), lambda i: (0, 0)),
        pl.BlockSpec((D, D), lambda i: (0, 0)),
        pl.BlockSpec((1, D), lambda i: (0, 0)),
    ],
    out_specs=pl.BlockSpec((_BLK, D), lambda i: (i, 0)),
    out_shape=jax.ShapeDtypeStruct((N_NODES, D), jnp.float32),
)


def kernel(x, edge_index, edge_attr, W_rel1, b_rel1, W_root1,
           W_rel2, b_rel2, W_root2):
    src = edge_index[0]
    dst = edge_index[1]
    zeros = jnp.zeros((N_NODES, D), jnp.float32)

    p1 = _sc_agg(x, src, dst, edge_attr, zeros)
    h1 = _tc_combine(p1, x, W_rel1, W_root1, b_rel1.reshape(1, D))
    p2 = _sc_agg(h1, src, dst, edge_attr, zeros)
    h2 = _tc_combine(p2, h1, W_rel2, W_root2, b_rel2.reshape(1, D))
    return h2
